# Initial kernel scaffold; baseline (speedup 1.0000x reference)
import jax
import jax.numpy as jnp
from jax.experimental import pallas as pl


def kernel(h, edge_index, W_msg, W_upd, b_upd):
    raise NotImplementedError("write your pallas kernel here")


# speedup vs baseline: 7.4935x; 7.4935x over previous
import functools

import jax
import jax.numpy as jnp
from jax import lax
from jax.experimental import pallas as pl
from jax.experimental.pallas import tpu as pltpu
from jax.experimental.pallas import tpu_sc as plsc

_NC = 2
_NS = 16
_NW = _NC * _NS
_CH = 80


def _sc_aggregate(h, src_r, dst_r):
    N, D = h.shape
    _, NCH, CH = src_r.shape
    NPAD = 10240
    RPT = NPAD // _NS
    ZR = 32

    mesh = plsc.VectorSubcoreMesh(core_axis_name="c", subcore_axis_name="s")

    @functools.partial(
        pl.kernel,
        out_type=jax.ShapeDtypeStruct((_NC, N, D), jnp.float32),
        mesh=mesh,
        scratch_types=[
            pltpu.VMEM_SHARED((NPAD, D), jnp.float32),
            pltpu.VMEM((NCH, CH), jnp.int32),
            pltpu.VMEM((NCH, CH), jnp.int32),
            pltpu.VMEM((CH, D), jnp.float32),
            pltpu.VMEM((ZR, D), jnp.float32),
            pltpu.SemaphoreType.DMA,
        ],
    )
    def agg_kernel(h_hbm, src_hbm, dst_hbm, out_hbm,
                   acc, src_v, dst_v, rows_v, zbuf, sem):
        c = lax.axis_index("c")
        s = lax.axis_index("s")
        wid = c * _NS + s

        zero = jnp.zeros((16,), jnp.float32)
        for i in range(ZR):
            for j in range(D // 16):
                zbuf[i, pl.ds(j * 16, 16)] = zero
        for k in range(RPT // ZR):
            pltpu.sync_copy(zbuf, acc.at[pl.ds(s * RPT + k * ZR, ZR)])
        plsc.subcore_barrier()

        pltpu.sync_copy(src_hbm.at[wid], src_v)
        pltpu.sync_copy(dst_hbm.at[wid], dst_v)

        def chunk(i, carry):
            pltpu.async_copy(h_hbm.at[src_v.at[i]], rows_v, sem).wait()
            pltpu.sync_copy(rows_v, acc.at[dst_v.at[i]], add=True)
            return carry

        lax.fori_loop(0, NCH, chunk, 0)
        plsc.subcore_barrier()

        last_full = N - (_NS - 1) * RPT

        @pl.when(s < _NS - 1)
        def _():
            pltpu.sync_copy(acc.at[pl.ds(s * RPT, RPT)],
                            out_hbm.at[c, pl.ds(s * RPT, RPT)])

        @pl.when(s == _NS - 1)
        def _():
            pltpu.sync_copy(acc.at[pl.ds((_NS - 1) * RPT, last_full)],
                            out_hbm.at[c, pl.ds((_NS - 1) * RPT, last_full)])

    return agg_kernel(h, src_r, dst_r)


def _dense(h, parts, W_msg, Wu_h, Wu_a, b):
    N, D = h.shape
    BLK = 400
    dn = (((1,), (1,)), ((), ()))

    def body(h_ref, p_ref, wm_ref, wh_ref, wa_ref, b_ref, o_ref):
        g = p_ref[0] + p_ref[1]
        agg = lax.dot_general(g, wm_ref[...], dn,
                              preferred_element_type=jnp.float32)
        acc = lax.dot_general(h_ref[...], wh_ref[...], dn,
                              preferred_element_type=jnp.float32)
        acc = acc + lax.dot_general(agg, wa_ref[...], dn,
                                    preferred_element_type=jnp.float32)
        o_ref[...] = jnp.maximum(acc + b_ref[...], 0.0)

    return pl.pallas_call(
        body,
        grid=(N // BLK,),
        in_specs=[
            pl.BlockSpec((BLK, D), lambda i: (i, 0)),
            pl.BlockSpec((_NC, BLK, D), lambda i: (0, i, 0)),
            pl.BlockSpec((D, D), lambda i: (0, 0)),
            pl.BlockSpec((D, D), lambda i: (0, 0)),
            pl.BlockSpec((D, D), lambda i: (0, 0)),
            pl.BlockSpec((1, D), lambda i: (0, 0)),
        ],
        out_specs=pl.BlockSpec((BLK, D), lambda i: (i, 0)),
        out_shape=jax.ShapeDtypeStruct((N, D), jnp.float32),
    )(h, parts, W_msg, Wu_h, Wu_a, b)


def kernel(h, edge_index, W_msg, W_upd, b_upd):
    N, D = h.shape
    E = edge_index.shape[1]
    nch = E // _NW // _CH
    src = edge_index[0].astype(jnp.int32).reshape(_NW, nch, _CH)
    dst = edge_index[1].astype(jnp.int32).reshape(_NW, nch, _CH)
    parts = _sc_aggregate(h, src, dst)
    return _dense(h, parts, W_msg, W_upd[:, :D], W_upd[:, D:],
                  b_upd.reshape(1, D))
